# baseline (speedup 1.0000x reference)
import functools

import jax
import jax.numpy as jnp
from jax import lax
from jax.experimental import pallas as pl
from jax.experimental.pallas import tpu as pltpu
from jax.experimental.pallas import tpu_sc as plsc

_NC = 2
_NS = 16
_NW = _NC * _NS
_NB = 4
_W = 16
_D = 14 * 128


def _table_body(res_ref, atom_ref, crgt_ref, wf_ref, bf_ref, out_ref):
    t_res = jnp.dot(res_ref[...], wf_ref[0:64, :], preferred_element_type=jnp.float32)
    t_atom = jnp.dot(atom_ref[...], wf_ref[64:128, :], preferred_element_type=jnp.float32)
    t_crg = jnp.dot(crgt_ref[...], wf_ref[128:144, :], preferred_element_type=jnp.float32)
    row_r = jax.lax.broadcasted_iota(jnp.int32, (63, 21), 0) // 3
    col_r = jax.lax.broadcasted_iota(jnp.int32, (63, 21), 1)
    e_res = (col_r == row_r).astype(jnp.float32)
    row_c = jax.lax.broadcasted_iota(jnp.int32, (63, 3), 0) % 3
    col_c = jax.lax.broadcasted_iota(jnp.int32, (63, 3), 1)
    e_crg = (col_c == row_c).astype(jnp.float32)
    rows63 = (jnp.dot(e_res, t_res, preferred_element_type=jnp.float32)
              + jnp.dot(e_crg, t_crg, preferred_element_type=jnp.float32)
              + bf_ref[...][None, :])
    for a in range(14):
        out_ref[:, a * 128:(a + 1) * 128] = rows63 + t_atom[a:a + 1, :]


def _build_t3(res_table, atom_table, crg_table, Wf, bf):
    full = lambda shape: pl.BlockSpec(shape, lambda: (0,) * len(shape))
    return pl.pallas_call(
        _table_body,
        in_specs=[full((21, 64)), full((14, 64)), full((3, 16)),
                  full((144, 128)), full((128,))],
        out_specs=full((63, _D)),
        out_shape=jax.ShapeDtypeStruct((63, _D), jnp.float32),
    )(res_table, atom_table, crg_table, Wf, bf)


def _sc_gather(t3, aa_flat, crg_flat, nl):
    b_per_w = nl // _NW
    n_chunks = b_per_w // _W
    mesh = plsc.VectorSubcoreMesh(core_axis_name="c", subcore_axis_name="s")

    @functools.partial(
        pl.kernel, mesh=mesh,
        out_type=jax.ShapeDtypeStruct((nl, _D), jnp.float32),
        scratch_types=(
            [pltpu.VMEM((b_per_w,), jnp.int32),
             pltpu.VMEM((b_per_w,), jnp.int32)]
            + [pltpu.VMEM((_W, _D), jnp.float32) for _ in range(_NB)]
            + [pltpu.SemaphoreType.DMA for _ in range(2 * _NB)]
        ),
    )
    def sc_kernel(t3_hbm, aa_hbm, crg_hbm, out_hbm, aa_v, idx_v, *bufsem):
        bufs = bufsem[:_NB]
        gsem = bufsem[_NB:2 * _NB]
        wsem = bufsem[2 * _NB:]
        wid = lax.axis_index("s") * _NC + lax.axis_index("c")
        base = wid * b_per_w
        pltpu.sync_copy(aa_hbm.at[pl.ds(base, b_per_w)], aa_v)
        pltpu.sync_copy(crg_hbm.at[pl.ds(base, b_per_w)], idx_v)

        @pl.loop(0, b_per_w // 16)
        def _(i):
            s = pl.ds(i * 16, 16)
            idx_v[s] = aa_v[s] * 3 + idx_v[s]

        def gather(c, b):
            return pltpu.make_async_copy(
                t3_hbm.at[idx_v.at[pl.ds(c * _W, _W)]], bufs[b], gsem[b])

        def wback(c, b):
            return pltpu.make_async_copy(
                bufs[b], out_hbm.at[pl.ds(base + c * _W, _W)], wsem[b])

        for k in range(_NB):
            gather(k, k).start()

        @pl.loop(0, n_chunks, step=_NB)
        def _(c):
            for b in range(_NB):
                cc = c + b
                gather(cc, b).wait()
                wback(cc, b).start()

                @pl.when(cc + _NB < n_chunks)
                def _(cc=cc, b=b):
                    wback(cc, b).wait()
                    gather(cc + _NB, b).start()

        for b in range(_NB):
            cc = n_chunks - _NB + b
            wback(cc, b).wait()

    return sc_kernel(t3, aa_flat, crg_flat)


def kernel(aa, pos14, atom_mask, phys, crg, res_table, atom_table, crg_table, Wp, bp, Wf, bf):
    N, L = aa.shape
    NL = N * L
    t3 = _build_t3(res_table, atom_table, crg_table, Wf, bf)
    out = _sc_gather(t3, aa.reshape(NL), crg.reshape(NL), NL)
    feats = out.reshape(N, L * 14, 128)
    coors = pos14.reshape(N, L * 14, 3)
    mask = atom_mask.reshape(N, L * 14)
    return (feats, coors, mask)


# speedup vs baseline: 1.7420x; 1.7420x over previous
import functools

import jax
import jax.numpy as jnp
from jax import lax
from jax.experimental import pallas as pl
from jax.experimental.pallas import tpu as pltpu
from jax.experimental.pallas import tpu_sc as plsc

_NC = 2
_NS = 16
_NW = _NC * _NS
_NB = 4
_W = 16


def _table_body(res_ref, atom_ref, crgt_ref, wf_ref, bf_ref, out_ref):
    t_res = jnp.dot(res_ref[...], wf_ref[0:64, :], preferred_element_type=jnp.float32)
    t_atom = jnp.dot(atom_ref[...], wf_ref[64:128, :], preferred_element_type=jnp.float32)
    t_crg = jnp.dot(crgt_ref[...], wf_ref[128:144, :], preferred_element_type=jnp.float32)
    row8 = jax.lax.broadcasted_iota(jnp.int32, (896, 21), 0)
    col8 = jax.lax.broadcasted_iota(jnp.int32, (896, 21), 1)
    e_res8 = (col8 == jnp.minimum(row8 // 42, 20)).astype(jnp.float32)
    e_crg8 = (col8[:, :3] == (row8[:, :3] // 14) % 3).astype(jnp.float32)
    e_atom8 = (col8[:, :14] == row8[:, :14] % 14).astype(jnp.float32)
    out_ref[...] = (jnp.dot(e_res8, t_res, preferred_element_type=jnp.float32)
                    + jnp.dot(e_crg8, t_crg, preferred_element_type=jnp.float32)
                    + jnp.dot(e_atom8, t_atom, preferred_element_type=jnp.float32)
                    + bf_ref[...][None, :])


def _build_t3(res_table, atom_table, crg_table, Wf, bf):
    full = lambda shape: pl.BlockSpec(shape, lambda: (0,) * len(shape))
    return pl.pallas_call(
        _table_body,
        in_specs=[full((21, 64)), full((14, 64)), full((3, 16)),
                  full((144, 128)), full((128,))],
        out_specs=full((896, 128)),
        out_shape=jax.ShapeDtypeStruct((896, 128), jnp.float32),
    )(res_table, atom_table, crg_table, Wf, bf)


def _sc_gather(t3, aa_flat, crg_flat, nl):
    b_per_w = nl // _NW
    n_chunks = b_per_w // _W
    mesh = plsc.VectorSubcoreMesh(core_axis_name="c", subcore_axis_name="s")

    @functools.partial(
        pl.kernel, mesh=mesh,
        out_type=jax.ShapeDtypeStruct((nl * 14, 128), jnp.float32),
        scratch_types=(
            [pltpu.HBM((63, 14, 128), jnp.float32),
             pltpu.VMEM((112, 128), jnp.float32),
             pltpu.VMEM((b_per_w,), jnp.int32),
             pltpu.VMEM((b_per_w,), jnp.int32)]
            + [pltpu.VMEM((_W * 14, 128), jnp.float32) for _ in range(_NB)]
            + [pltpu.SemaphoreType.DMA for _ in range(2 * _NB)]
        ),
    )
    def sc_kernel(t3_hbm, aa_hbm, crg_hbm, out_hbm, t3s, stage_v, aa_v, idx_v, *bufsem):
        bufs = bufsem[:_NB]
        gsem = bufsem[_NB:2 * _NB]
        wsem = bufsem[2 * _NB:]
        wid = lax.axis_index("s") * _NC + lax.axis_index("c")
        base = wid * b_per_w

        @pl.when(lax.axis_index("s") == 0)
        def _():
            @pl.loop(0, 8)
            def _(j):
                pltpu.sync_copy(t3_hbm.at[pl.ds(j * 112, 112)], stage_v)
                pltpu.sync_copy(stage_v.reshape(8, 14, 128),
                                t3s.at[pl.ds(j * 8, 8)])

        plsc.subcore_barrier()
        pltpu.sync_copy(aa_hbm.at[pl.ds(base, b_per_w)], aa_v)
        pltpu.sync_copy(crg_hbm.at[pl.ds(base, b_per_w)], idx_v)

        @pl.loop(0, b_per_w // 16)
        def _(i):
            s = pl.ds(i * 16, 16)
            idx_v[s] = aa_v[s] * 3 + idx_v[s]

        def gather(c, b):
            return pltpu.make_async_copy(
                t3s.at[idx_v.at[pl.ds(c * _W, _W)]],
                bufs[b].reshape(_W, 14, 128), gsem[b])

        def wback(c, b):
            return pltpu.make_async_copy(
                bufs[b],
                out_hbm.at[pl.ds((base + c * _W) * 14, _W * 14)],
                wsem[b])

        for k in range(_NB):
            gather(k, k).start()

        @pl.loop(0, n_chunks, step=_NB)
        def _(c):
            for b in range(_NB):
                cc = c + b
                gather(cc, b).wait()
                wback(cc, b).start()

                @pl.when(cc + _NB < n_chunks)
                def _(cc=cc, b=b):
                    wback(cc, b).wait()
                    gather(cc + _NB, b).start()

        for b in range(_NB):
            cc = n_chunks - _NB + b
            wback(cc, b).wait()

    return sc_kernel(t3, aa_flat, crg_flat)


def kernel(aa, pos14, atom_mask, phys, crg, res_table, atom_table, crg_table, Wp, bp, Wf, bf):
    N, L = aa.shape
    NL = N * L
    t3 = _build_t3(res_table, atom_table, crg_table, Wf, bf)
    out = _sc_gather(t3, aa.reshape(NL), crg.reshape(NL), NL)
    feats = out.reshape(N, L * 14, 128)
    coors = pos14.reshape(N, L * 14, 3)
    mask = atom_mask.reshape(N, L * 14)
    return (feats, coors, mask)
